# baseline (speedup 1.0000x reference)
import functools

import jax
import jax.numpy as jnp
from jax import lax
from jax.experimental import pallas as pl
from jax.experimental.pallas import tpu as pltpu
from jax.experimental.pallas import tpu_sc as plsc

N = 10000
D = 128
H = 128
O = 64

NC = 2
NS = 16
NW = NC * NS
C = 128

RPT = 632
NPAD = NS * RPT
DEGW = 128

_mesh = plsc.VectorSubcoreMesh(core_axis_name="c", subcore_axis_name="s")

SPLIT_NUM0 = 0
SPLIT_DEN = 1


def _zero_vmem(buf, rows, width):
    def body(i, _):
        for j in range(width // 16):
            buf[i, pl.ds(j * 16, 16)] = jnp.zeros((16,), jnp.float32)
        return 0
    lax.fori_loop(0, rows, body, 0)


def _acc_zero_and_barrier(zbuf, acc, s):
    base = s * RPT
    for off in (0, 128, 256, 384, RPT - C):
        pltpu.sync_copy(zbuf, acc.at[pl.ds(base + off, C)])
    plsc.subcore_barrier()


def _make_deg_kernel(epad):
    nchunk = epad // (NW * C)
    epw = nchunk * C

    @functools.partial(
        pl.kernel,
        mesh=_mesh,
        out_type=jax.ShapeDtypeStruct((2 * NPAD, DEGW), jnp.float32),
        scratch_types=[
            pltpu.VMEM((C,), jnp.int32),
            pltpu.VMEM((C, DEGW), jnp.float32),
            pltpu.VMEM((C, DEGW), jnp.float32),
            pltpu.VMEM_SHARED((NPAD, DEGW), jnp.float32),
        ],
    )
    def deg_kernel(dst_hbm, ones_hbm, out_hbm, didx, ones, zbuf, acc):
        c = lax.axis_index("c")
        s = lax.axis_index("s")
        _zero_vmem(zbuf, C, DEGW)
        pltpu.sync_copy(ones_hbm, ones)
        _acc_zero_and_barrier(zbuf, acc, s)

        ebase = (c * NS + s) * epw
        def body(j, _):
            pltpu.sync_copy(dst_hbm.at[pl.ds(ebase + j * C, C)], didx)
            pltpu.sync_copy(ones, acc.at[didx], add=True)
            return 0
        lax.fori_loop(0, nchunk, body, 0)

        plsc.subcore_barrier()
        pltpu.sync_copy(
            acc.at[pl.ds(s * RPT, RPT)],
            out_hbm.at[pl.ds(c * NPAD + s * RPT, RPT)],
        )

    return deg_kernel


def _make_scatter_kernel(epad, n0, n1):
    nchunk = epad // (NW * C)
    assert nchunk % 2 == 0
    assert n0 + n1 == 2 * nchunk and n0 % 2 == 0 and n1 % 2 == 0

    NSB = 64
    nsbs = (max(n0, n1) + NSB - 1) // NSB

    @functools.partial(
        pl.kernel,
        mesh=_mesh,
        out_type=jax.ShapeDtypeStruct((2 * NPAD, H), jnp.float32),
        scratch_types=[
            pltpu.VMEM((NSB, C), jnp.int32),
            pltpu.VMEM((NSB, C), jnp.int32),
            pltpu.VMEM((C, H), jnp.float32),
            pltpu.VMEM((C, H), jnp.float32),
            pltpu.VMEM_SHARED((NPAD, H), jnp.float32),
            pltpu.SemaphoreType.DMA,
            pltpu.SemaphoreType.DMA,
        ],
    )
    def scatter_kernel(hs_hbm, src_hbm, dst_hbm, out_hbm,
                       sidx, didx, rows0, rows1, acc, sem0, sem1):
        c = lax.axis_index("c")
        s = lax.axis_index("s")
        _zero_vmem(rows0, C, H)
        _acc_zero_and_barrier(rows0, acc, s)

        nc = jnp.where(c == 0, n0, n1)
        cbase = jnp.where(c == 0, s * n0, NS * n0 + s * n1)

        for sb in range(nsbs):
            sbn = jnp.clip(nc - sb * NSB, 0, NSB)

            @pl.when(sbn > 0)
            def _():
                pltpu.sync_copy(src_hbm.at[pl.ds(cbase + sb * NSB, NSB)], sidx)
                pltpu.sync_copy(dst_hbm.at[pl.ds(cbase + sb * NSB, NSB)], didx)

                pltpu.async_copy(hs_hbm.at[sidx.at[0]], rows0, sem0)
                pltpu.async_copy(hs_hbm.at[sidx.at[1]], rows1, sem1)

                def body(p, _):
                    j0 = 2 * p
                    pltpu.make_async_copy(hs_hbm.at[sidx.at[j0]], rows0, sem0).wait()
                    pltpu.sync_copy(rows0, acc.at[didx.at[j0]], add=True)
                    @pl.when(p < sbn // 2 - 1)
                    def _():
                        pltpu.async_copy(hs_hbm.at[sidx.at[j0 + 2]], rows0, sem0)
                    pltpu.make_async_copy(hs_hbm.at[sidx.at[j0 + 1]], rows1, sem1).wait()
                    pltpu.sync_copy(rows1, acc.at[didx.at[j0 + 1]], add=True)
                    @pl.when(p < sbn // 2 - 1)
                    def _():
                        pltpu.async_copy(hs_hbm.at[sidx.at[j0 + 3]], rows1, sem1)
                    return 0
                lax.fori_loop(0, sbn // 2, body, 0)

        plsc.subcore_barrier()
        pltpu.sync_copy(
            acc.at[pl.ds(s * RPT, RPT)],
            out_hbm.at[pl.ds(c * NPAD + s * RPT, RPT)],
        )

    return scatter_kernel


def _k1_body(degp, x, w1, dinv_ref, hs1_ref):
    deg = degp[0:N, 0:1] + degp[NPAD:NPAD + N, 0:1] + 1.0
    dinv = lax.rsqrt(jnp.maximum(deg, 1.0))
    dinv_ref[...] = dinv
    hs1_ref[...] = (
        jnp.dot(x[...], w1[...], preferred_element_type=jnp.float32,
                precision=lax.Precision.HIGHEST) * dinv
    )


def _klayer_body(aggp, hs, dinv, b, g, be, w, out_ref):
    t = (aggp[0:N, :] + aggp[NPAD:NPAD + N, :] + hs[...]) * dinv[...] + b[...]
    r = jnp.maximum(t, 0.0)
    m = jnp.mean(r, axis=0, keepdims=True)
    cen = r - m
    v = jnp.mean(cen * cen, axis=0, keepdims=True)
    hn = cen * lax.rsqrt(v + 1e-5) * g[...] + be[...]
    out_ref[...] = (
        jnp.dot(hn, w[...], preferred_element_type=jnp.float32,
                precision=lax.Precision.HIGHEST) * dinv[...]
    )


def _kfinal_body(aggp, hs, dinv, b, g, be, wf, bf, out_ref):
    t = (aggp[0:N, :] + aggp[NPAD:NPAD + N, :] + hs[...]) * dinv[...] + b[...]
    r = jnp.maximum(t, 0.0)
    m = jnp.mean(r, axis=0, keepdims=True)
    cen = r - m
    v = jnp.mean(cen * cen, axis=0, keepdims=True)
    hn = cen * lax.rsqrt(v + 1e-5) * g[...] + be[...]
    out_ref[...] = (
        jnp.dot(hn, wf[...], preferred_element_type=jnp.float32,
                precision=lax.Precision.HIGHEST) + bf[...]
    )


def kernel(x, edge_index, W1, b1, g1, be1, W2, b2, g2, be2,
           W3, b3, g3, be3, Wf, bf):
    E = edge_index.shape[1]
    epad = ((E + 2 * NW * C - 1) // (2 * NW * C)) * (2 * NW * C)
    pad = epad - E + 64 * C
    src = edge_index[0]
    dst = edge_index[1]
    src = jnp.concatenate([src, jnp.zeros((pad,), jnp.int32)])
    dst = jnp.concatenate([dst, jnp.full((pad,), N, jnp.int32)])
    nchunk = epad // (NW * C)
    src3 = src.reshape(-1, C)
    dst3 = dst.reshape(-1, C)

    n0 = (SPLIT_NUM0 * 2 * nchunk // SPLIT_DEN // 2) * 2
    n1 = 2 * nchunk - n0
    deg_k = _make_deg_kernel(epad)
    scat_k = _make_scatter_kernel(epad, n0, n1)

    degp = deg_k(dst, jnp.ones((C, DEGW), jnp.float32))

    k1 = pl.pallas_call(
        _k1_body,
        out_shape=[
            jax.ShapeDtypeStruct((N, 1), jnp.float32),
            jax.ShapeDtypeStruct((N, H), jnp.float32),
        ],
    )
    dinv, hs1 = k1(degp, x, W1)

    klayer = pl.pallas_call(
        _klayer_body,
        out_shape=jax.ShapeDtypeStruct((N, H), jnp.float32),
    )
    kfinal = pl.pallas_call(
        _kfinal_body,
        out_shape=jax.ShapeDtypeStruct((N, O), jnp.float32),
    )

    b1r = b1.reshape(1, H)
    g1r = g1.reshape(1, H)
    be1r = be1.reshape(1, H)
    b2r = b2.reshape(1, H)
    g2r = g2.reshape(1, H)
    be2r = be2.reshape(1, H)
    b3r = b3.reshape(1, H)
    g3r = g3.reshape(1, H)
    be3r = be3.reshape(1, H)
    bfr = bf.reshape(1, O)

    aggp1 = scat_k(hs1, src3, dst3)
    hs2 = klayer(aggp1, hs1, dinv, b1r, g1r, be1r, W2)
    aggp2 = scat_k(hs2, src3, dst3)
    hs3 = klayer(aggp2, hs2, dinv, b2r, g2r, be2r, W3)
    aggp3 = scat_k(hs3, src3, dst3)
    out = kfinal(aggp3, hs3, dinv, b3r, g3r, be3r, Wf, bfr)
    return out


# speedup vs baseline: 1.3197x; 1.3197x over previous
import functools

import jax
import jax.numpy as jnp
from jax import lax
from jax.experimental import pallas as pl
from jax.experimental.pallas import tpu as pltpu
from jax.experimental.pallas import tpu_sc as plsc

N = 10000
D = 128
H = 128
O = 64

NC = 2
NS = 16
NW = NC * NS
C = 128

RPT = 632
NPAD = NS * RPT
DEGW = 128

_mesh = plsc.VectorSubcoreMesh(core_axis_name="c", subcore_axis_name="s")


def _zero_vmem(buf, rows, width):
    def body(i, _):
        for j in range(width // 16):
            buf[i, pl.ds(j * 16, 16)] = jnp.zeros((16,), jnp.float32)
        return 0
    lax.fori_loop(0, rows, body, 0)


def _acc_zero_and_barrier(zbuf, acc, s):
    base = s * RPT
    for off in (0, 128, 256, 384, RPT - C):
        pltpu.sync_copy(zbuf, acc.at[pl.ds(base + off, C)])
    plsc.subcore_barrier()


def _make_deg_kernel(epad):
    nchunk = epad // (NW * C)
    epw = nchunk * C

    @functools.partial(
        pl.kernel,
        mesh=_mesh,
        out_type=jax.ShapeDtypeStruct((2 * NPAD, DEGW), jnp.float32),
        scratch_types=[
            pltpu.VMEM((C,), jnp.int32),
            pltpu.VMEM((C, DEGW), jnp.float32),
            pltpu.VMEM((C, DEGW), jnp.float32),
            pltpu.VMEM_SHARED((NPAD, DEGW), jnp.float32),
        ],
    )
    def deg_kernel(dst_hbm, ones_hbm, out_hbm, didx, ones, zbuf, acc):
        c = lax.axis_index("c")
        s = lax.axis_index("s")
        _zero_vmem(zbuf, C, DEGW)
        pltpu.sync_copy(ones_hbm, ones)
        _acc_zero_and_barrier(zbuf, acc, s)

        ebase = (c * NS + s) * epw
        def body(j, _):
            pltpu.sync_copy(dst_hbm.at[pl.ds(ebase + j * C, C)], didx)
            pltpu.sync_copy(ones, acc.at[didx], add=True)
            return 0
        lax.fori_loop(0, nchunk, body, 0)

        plsc.subcore_barrier()
        pltpu.sync_copy(
            acc.at[pl.ds(s * RPT, RPT)],
            out_hbm.at[pl.ds(c * NPAD + s * RPT, RPT)],
        )

    return deg_kernel


def _make_scatter_kernel(epad):
    nchunk = epad // (NW * C)
    epw = nchunk * C

    @functools.partial(
        pl.kernel,
        mesh=_mesh,
        out_type=jax.ShapeDtypeStruct((2 * NPAD, H), jnp.float32),
        scratch_types=[
            pltpu.VMEM((C,), jnp.int32),
            pltpu.VMEM((C,), jnp.int32),
            pltpu.VMEM((C, H), jnp.float32),
            pltpu.VMEM((C, H), jnp.float32),
            pltpu.VMEM_SHARED((NPAD, H), jnp.float32),
            pltpu.SemaphoreType.DMA,
        ],
    )
    def scatter_kernel(hs_hbm, src_hbm, dst_hbm, out_hbm,
                       sidx, didx, rows, zbuf, acc, sem):
        c = lax.axis_index("c")
        s = lax.axis_index("s")
        _zero_vmem(zbuf, C, H)
        _acc_zero_and_barrier(zbuf, acc, s)

        ebase = (c * NS + s) * epw
        def body(j, _):
            off = ebase + j * C
            pltpu.sync_copy(src_hbm.at[pl.ds(off, C)], sidx)
            pltpu.sync_copy(dst_hbm.at[pl.ds(off, C)], didx)
            pltpu.async_copy(hs_hbm.at[sidx], rows, sem).wait()
            pltpu.sync_copy(rows, acc.at[didx], add=True)
            return 0
        lax.fori_loop(0, nchunk, body, 0)

        plsc.subcore_barrier()
        pltpu.sync_copy(
            acc.at[pl.ds(s * RPT, RPT)],
            out_hbm.at[pl.ds(c * NPAD + s * RPT, RPT)],
        )

    return scatter_kernel


def _k1_body(degp, x, w1, dinv_ref, hs1_ref):
    deg = degp[0:N, 0:1] + degp[NPAD:NPAD + N, 0:1] + 1.0
    dinv = lax.rsqrt(jnp.maximum(deg, 1.0))
    dinv_ref[...] = dinv
    hs1_ref[...] = (
        jnp.dot(x[...], w1[...], preferred_element_type=jnp.float32,
                precision=lax.Precision.HIGHEST) * dinv
    )


def _klayer_body(aggp, hs, dinv, b, g, be, w, out_ref):
    t = (aggp[0:N, :] + aggp[NPAD:NPAD + N, :] + hs[...]) * dinv[...] + b[...]
    r = jnp.maximum(t, 0.0)
    m = jnp.mean(r, axis=0, keepdims=True)
    cen = r - m
    v = jnp.mean(cen * cen, axis=0, keepdims=True)
    hn = cen * lax.rsqrt(v + 1e-5) * g[...] + be[...]
    out_ref[...] = (
        jnp.dot(hn, w[...], preferred_element_type=jnp.float32,
                precision=lax.Precision.HIGHEST) * dinv[...]
    )


def _kfinal_body(aggp, hs, dinv, b, g, be, wf, bf, out_ref):
    t = (aggp[0:N, :] + aggp[NPAD:NPAD + N, :] + hs[...]) * dinv[...] + b[...]
    r = jnp.maximum(t, 0.0)
    m = jnp.mean(r, axis=0, keepdims=True)
    cen = r - m
    v = jnp.mean(cen * cen, axis=0, keepdims=True)
    hn = cen * lax.rsqrt(v + 1e-5) * g[...] + be[...]
    out_ref[...] = (
        jnp.dot(hn, wf[...], preferred_element_type=jnp.float32,
                precision=lax.Precision.HIGHEST) + bf[...]
    )


def kernel(x, edge_index, W1, b1, g1, be1, W2, b2, g2, be2,
           W3, b3, g3, be3, Wf, bf):
    E = edge_index.shape[1]
    epad = ((E + NW * C - 1) // (NW * C)) * (NW * C)
    pad = epad - E
    src = edge_index[0]
    dst = edge_index[1]
    if pad:
        src = jnp.concatenate([src, jnp.zeros((pad,), jnp.int32)])
        dst = jnp.concatenate([dst, jnp.full((pad,), N, jnp.int32)])

    deg_k = _make_deg_kernel(epad)
    scat_k = _make_scatter_kernel(epad)

    degp = deg_k(dst, jnp.ones((C, DEGW), jnp.float32))

    k1 = pl.pallas_call(
        _k1_body,
        out_shape=[
            jax.ShapeDtypeStruct((N, 1), jnp.float32),
            jax.ShapeDtypeStruct((N, H), jnp.float32),
        ],
    )
    dinv, hs1 = k1(degp, x, W1)

    klayer = pl.pallas_call(
        _klayer_body,
        out_shape=jax.ShapeDtypeStruct((N, H), jnp.float32),
    )
    kfinal = pl.pallas_call(
        _kfinal_body,
        out_shape=jax.ShapeDtypeStruct((N, O), jnp.float32),
    )

    b1r = b1.reshape(1, H)
    g1r = g1.reshape(1, H)
    be1r = be1.reshape(1, H)
    b2r = b2.reshape(1, H)
    g2r = g2.reshape(1, H)
    be2r = be2.reshape(1, H)
    b3r = b3.reshape(1, H)
    g3r = g3.reshape(1, H)
    be3r = be3.reshape(1, H)
    bfr = bf.reshape(1, O)

    aggp1 = scat_k(hs1, src, dst)
    hs2 = klayer(aggp1, hs1, dinv, b1r, g1r, be1r, W2)
    aggp2 = scat_k(hs2, src, dst)
    hs3 = klayer(aggp2, hs2, dinv, b2r, g2r, be2r, W3)
    aggp3 = scat_k(hs3, src, dst)
    out = kfinal(aggp3, hs3, dinv, b3r, g3r, be3r, Wf, bfr)
    return out


# speedup vs baseline: 1.4358x; 1.0880x over previous
import functools

import jax
import jax.numpy as jnp
from jax import lax
from jax.experimental import pallas as pl
from jax.experimental.pallas import tpu as pltpu
from jax.experimental.pallas import tpu_sc as plsc

N = 10000
D = 128
H = 128
O = 64

NC = 2
NS = 16
NW = NC * NS
C = 128

RPT = 632
NPAD = NS * RPT
DEGW = 128

_mesh = plsc.VectorSubcoreMesh(core_axis_name="c", subcore_axis_name="s")


def _zero_vmem(buf, rows, width):
    def body(i, _):
        for j in range(width // 16):
            buf[i, pl.ds(j * 16, 16)] = jnp.zeros((16,), jnp.float32)
        return 0
    lax.fori_loop(0, rows, body, 0)


def _acc_zero_and_barrier(zbuf, acc, s):
    base = s * RPT
    for off in (0, 128, 256, 384, RPT - C):
        pltpu.sync_copy(zbuf, acc.at[pl.ds(base + off, C)])
    plsc.subcore_barrier()


def _make_deg_kernel(epad):
    nchunk = epad // (NW * C)
    epw = nchunk * C

    @functools.partial(
        pl.kernel,
        mesh=_mesh,
        out_type=jax.ShapeDtypeStruct((2 * NPAD, DEGW), jnp.float32),
        scratch_types=[
            pltpu.VMEM((C,), jnp.int32),
            pltpu.VMEM((C, DEGW), jnp.float32),
            pltpu.VMEM((C, DEGW), jnp.float32),
            pltpu.VMEM_SHARED((NPAD, DEGW), jnp.float32),
        ],
    )
    def deg_kernel(dst_hbm, ones_hbm, out_hbm, didx, ones, zbuf, acc):
        c = lax.axis_index("c")
        s = lax.axis_index("s")
        _zero_vmem(zbuf, C, DEGW)
        pltpu.sync_copy(ones_hbm, ones)
        _acc_zero_and_barrier(zbuf, acc, s)

        ebase = (c * NS + s) * epw
        def body(j, _):
            pltpu.sync_copy(dst_hbm.at[pl.ds(ebase + j * C, C)], didx)
            pltpu.sync_copy(ones, acc.at[didx], add=True)
            return 0
        lax.fori_loop(0, nchunk, body, 0)

        plsc.subcore_barrier()
        pltpu.sync_copy(
            acc.at[pl.ds(s * RPT, RPT)],
            out_hbm.at[pl.ds(c * NPAD + s * RPT, RPT)],
        )

    return deg_kernel


def _make_scatter_kernel(epad):
    nchunk = epad // (NW * C)
    epw = nchunk * C

    @functools.partial(
        pl.kernel,
        mesh=_mesh,
        out_type=jax.ShapeDtypeStruct((2 * NPAD, H), jnp.float32),
        scratch_types=[
            pltpu.VMEM((2, C), jnp.int32),
            pltpu.VMEM((C, H), jnp.float32),
            pltpu.VMEM((C, H), jnp.float32),
            pltpu.VMEM_SHARED((NPAD, H), jnp.float32),
            pltpu.SemaphoreType.DMA,
        ],
    )
    def scatter_kernel(hs_hbm, sd_hbm, out_hbm,
                       sdidx, rows, zbuf, acc, sem):
        c = lax.axis_index("c")
        s = lax.axis_index("s")
        _zero_vmem(zbuf, C, H)
        _acc_zero_and_barrier(zbuf, acc, s)

        cbase = (c * NS + s) * nchunk
        def body(j, _):
            pltpu.sync_copy(sd_hbm.at[cbase + j], sdidx)
            pltpu.async_copy(hs_hbm.at[sdidx.at[0]], rows, sem).wait()
            pltpu.sync_copy(rows, acc.at[sdidx.at[1]], add=True)
            return 0
        lax.fori_loop(0, nchunk, body, 0)

        plsc.subcore_barrier()
        pltpu.sync_copy(
            acc.at[pl.ds(s * RPT, RPT)],
            out_hbm.at[pl.ds(c * NPAD + s * RPT, RPT)],
        )

    return scatter_kernel


def _k1_body(degp, x, w1, dinv_ref, hs1_ref):
    deg = degp[0:N, 0:1] + degp[NPAD:NPAD + N, 0:1] + 1.0
    dinv = lax.rsqrt(jnp.maximum(deg, 1.0))
    dinv_ref[...] = dinv
    hs1_ref[...] = (
        jnp.dot(x[...], w1[...], preferred_element_type=jnp.float32,
                precision=lax.Precision.HIGHEST) * dinv
    )


def _klayer_body(aggp, hs, dinv, b, g, be, w, out_ref):
    t = (aggp[0:N, :] + aggp[NPAD:NPAD + N, :] + hs[...]) * dinv[...] + b[...]
    r = jnp.maximum(t, 0.0)
    m = jnp.mean(r, axis=0, keepdims=True)
    cen = r - m
    v = jnp.mean(cen * cen, axis=0, keepdims=True)
    hn = cen * lax.rsqrt(v + 1e-5) * g[...] + be[...]
    out_ref[...] = (
        jnp.dot(hn, w[...], preferred_element_type=jnp.float32,
                precision=lax.Precision.HIGHEST) * dinv[...]
    )


def _kfinal_body(aggp, hs, dinv, b, g, be, wf, bf, out_ref):
    t = (aggp[0:N, :] + aggp[NPAD:NPAD + N, :] + hs[...]) * dinv[...] + b[...]
    r = jnp.maximum(t, 0.0)
    m = jnp.mean(r, axis=0, keepdims=True)
    cen = r - m
    v = jnp.mean(cen * cen, axis=0, keepdims=True)
    hn = cen * lax.rsqrt(v + 1e-5) * g[...] + be[...]
    out_ref[...] = (
        jnp.dot(hn, wf[...], preferred_element_type=jnp.float32,
                precision=lax.Precision.HIGHEST) + bf[...]
    )


def kernel(x, edge_index, W1, b1, g1, be1, W2, b2, g2, be2,
           W3, b3, g3, be3, Wf, bf):
    E = edge_index.shape[1]
    epad = ((E + NW * C - 1) // (NW * C)) * (NW * C)
    pad = epad - E
    src = edge_index[0]
    dst = edge_index[1]
    if pad:
        src = jnp.concatenate([src, jnp.zeros((pad,), jnp.int32)])
        dst = jnp.concatenate([dst, jnp.full((pad,), N, jnp.int32)])

    sd = jnp.stack([src.reshape(-1, C), dst.reshape(-1, C)], axis=1)

    deg_k = _make_deg_kernel(epad)
    scat_k = _make_scatter_kernel(epad)

    degp = deg_k(dst, jnp.ones((C, DEGW), jnp.float32))

    k1 = pl.pallas_call(
        _k1_body,
        out_shape=[
            jax.ShapeDtypeStruct((N, 1), jnp.float32),
            jax.ShapeDtypeStruct((N, H), jnp.float32),
        ],
    )
    dinv, hs1 = k1(degp, x, W1)

    klayer = pl.pallas_call(
        _klayer_body,
        out_shape=jax.ShapeDtypeStruct((N, H), jnp.float32),
    )
    kfinal = pl.pallas_call(
        _kfinal_body,
        out_shape=jax.ShapeDtypeStruct((N, O), jnp.float32),
    )

    b1r = b1.reshape(1, H)
    g1r = g1.reshape(1, H)
    be1r = be1.reshape(1, H)
    b2r = b2.reshape(1, H)
    g2r = g2.reshape(1, H)
    be2r = be2.reshape(1, H)
    b3r = b3.reshape(1, H)
    g3r = g3.reshape(1, H)
    be3r = be3.reshape(1, H)
    bfr = bf.reshape(1, O)

    aggp1 = scat_k(hs1, sd)
    hs2 = klayer(aggp1, hs1, dinv, b1r, g1r, be1r, W2)
    aggp2 = scat_k(hs2, sd)
    hs3 = klayer(aggp2, hs2, dinv, b2r, g2r, be2r, W3)
    aggp3 = scat_k(hs3, sd)
    out = kfinal(aggp3, hs3, dinv, b3r, g3r, be3r, Wf, bfr)
    return out
